# Initial kernel scaffold; baseline (speedup 1.0000x reference)
import jax
import jax.numpy as jnp
from jax.experimental import pallas as pl


def kernel(x, params):
    raise NotImplementedError("write your pallas kernel here")


# speedup vs baseline: 79.8809x; 79.8809x over previous
import functools

import jax
import jax.numpy as jnp
from jax.experimental import pallas as pl

_NA = 60
_K = 8
_INPUT_RADIUS = 0.4
_SIGMA = 0.5 * _INPUT_RADIUS ** 2
_N = 1024
_BIG = 1e30


def _leaky(v):
    return jnp.where(v >= 0, v, 0.01 * v)


def _fwd_kernel(x_ref, xt_ref, w0_ref, b0_ref, ws_ref, bs_ref,
                w1_ref, b1_ref, w2_ref, b2_ref, aq_ref,
                fcw_ref, fcb_ref, tw_ref, tb_ref, out_ref):
    x = x_ref[0]
    xt = xt_ref[0]

    d2 = None
    for c in range(3):
        diff = x[:, c:c + 1] - xt[c:c + 1, :]
        sq = diff * diff
        d2 = sq if d2 is None else d2 + sq

    lane = jax.lax.broadcasted_iota(jnp.int32, (_N, _N), 1)

    d2m = d2
    idxs = []
    dists = []
    for _ in range(_K + 1):
        m = jnp.min(d2m, axis=1, keepdims=True)
        hit = d2m == m
        ik = jnp.min(jnp.where(hit, lane, _N), axis=1, keepdims=True)
        d2m = jnp.where(lane == ik, _BIG, d2m)
        idxs.append(ik)
        dists.append(m)
    idxs = idxs[1:]
    dists = dists[1:]

    ws = [jnp.exp(-d / _SIGMA) for d in dists]
    s0 = functools.reduce(lambda a, b: a + b, ws)
    denom = s0 + 1e-8
    s = s0 / denom

    m_acc = jnp.zeros((_N, _N), jnp.float32)
    for k in range(_K):
        wn = ws[k] / denom
        m_acc = m_acc + jnp.where(lane == idxs[k], wn, 0.0)

    feat = _leaky(s * w0_ref[...] + b0_ref[...])
    for l in range(7):
        agg = jnp.dot(m_acc, feat, preferred_element_type=jnp.float32)
        feat = _leaky(jnp.dot(agg, ws_ref[l], preferred_element_type=jnp.float32)
                      + bs_ref[l:l + 1, :])

    h = jnp.maximum(jnp.dot(feat, w1_ref[...], preferred_element_type=jnp.float32)
                    + b1_ref[...], 0.0)
    h = jnp.dot(h, w2_ref[...], preferred_element_type=jnp.float32) + b2_ref[...]

    gfeat = jnp.max(h, axis=0, keepdims=True)
    fc = jnp.maximum(jnp.dot(gfeat, fcw_ref[...], preferred_element_type=jnp.float32)
                     + fcb_ref[...], 0.0)
    t_out = jnp.dot(fc, tw_ref[...], preferred_element_type=jnp.float32) + tb_ref[...]

    aq = aq_ref[...]
    q = jnp.mean(aq, axis=0, keepdims=True)
    qn = q / (jnp.sqrt(jnp.sum(q * q)) + 1e-8)

    i = pl.program_id(0)
    out_ref[pl.ds(i, 1), :] = jnp.concatenate([qn, t_out, fc], axis=1)


@jax.jit
def kernel(x, params):
    b = x.shape[0]
    xt = jnp.swapaxes(x, 1, 2)
    convs = params["convs"]
    w0 = convs[0][0]
    b0 = convs[0][1].reshape(1, 32)
    ws = jnp.stack([w for w, _ in convs[1:]])
    bs = jnp.stack([bb for _, bb in convs[1:]])

    fixed = lambda *zeros: (lambda i: zeros)
    out = pl.pallas_call(
        _fwd_kernel,
        out_shape=jax.ShapeDtypeStruct((b, 71), jnp.float32),
        grid=(b,),
        in_specs=[
            pl.BlockSpec((1, _N, 3), lambda i: (i, 0, 0)),
            pl.BlockSpec((1, 3, _N), lambda i: (i, 0, 0)),
            pl.BlockSpec((1, 32), fixed(0, 0)),
            pl.BlockSpec((1, 32), fixed(0, 0)),
            pl.BlockSpec((7, 32, 32), fixed(0, 0, 0)),
            pl.BlockSpec((7, 32), fixed(0, 0)),
            pl.BlockSpec((32, 128), fixed(0, 0)),
            pl.BlockSpec((1, 128), fixed(0, 0)),
            pl.BlockSpec((128, 128), fixed(0, 0)),
            pl.BlockSpec((1, 128), fixed(0, 0)),
            pl.BlockSpec((_NA, 4), fixed(0, 0)),
            pl.BlockSpec((128, 64), fixed(0, 0)),
            pl.BlockSpec((1, 64), fixed(0, 0)),
            pl.BlockSpec((64, 3), fixed(0, 0)),
            pl.BlockSpec((1, 3), fixed(0, 0)),
        ],
        out_specs=pl.BlockSpec((b, 71), lambda i: (0, 0)),
    )(x, xt, w0, b0, ws, bs,
      params["out_W1"], params["out_b1"].reshape(1, 128),
      params["out_W2"], params["out_b2"].reshape(1, 128),
      params["anchor_quats"],
      params["fc_W"], params["fc_b"].reshape(1, 64),
      params["t_W"], params["t_b"].reshape(1, 3))
    return out


# speedup vs baseline: 89.4890x; 1.1203x over previous
import functools

import jax
import jax.numpy as jnp
from jax.experimental import pallas as pl

_NA = 60
_K = 8
_INPUT_RADIUS = 0.4
_SIGMA = 0.5 * _INPUT_RADIUS ** 2
_N = 1024
_BIG = 1e30


def _leaky(v):
    return jnp.where(v >= 0, v, 0.01 * v)


def _fwd_kernel(x_ref, xt_ref, w0_ref, b0_ref, ws_ref, bs_ref,
                w1_ref, b1_ref, w2_ref, b2_ref, aq_ref,
                fcw_ref, fcb_ref, tw_ref, tb_ref, out_ref):
    x = x_ref[0]
    xt = xt_ref[0]

    d2 = None
    for c in range(3):
        diff = x[:, c:c + 1] - xt[c:c + 1, :]
        sq = diff * diff
        d2 = sq if d2 is None else d2 + sq

    lane = jax.lax.broadcasted_iota(jnp.int32, (_N, _N), 1)
    row = jax.lax.broadcasted_iota(jnp.int32, (_N, _N), 0)
    diag = lane == row

    d2m = jnp.where(diag, _BIG, d2)
    for _ in range(_K):
        m = jnp.min(d2m, axis=1, keepdims=True)
        ik = jnp.min(jnp.where(d2m == m, lane, _N), axis=1, keepdims=True)
        d2m = jnp.where(lane == ik, _BIG, d2m)

    nbr = (d2m == _BIG) & jnp.logical_not(diag)
    wfull = jnp.where(nbr, jnp.exp(d2 * (-1.0 / _SIGMA)), 0.0)
    s0 = jnp.sum(wfull, axis=1, keepdims=True)
    rden = 1.0 / (s0 + 1e-8)
    s = s0 * rden
    m_acc = wfull * rden

    feat = _leaky(s * w0_ref[...] + b0_ref[...])
    for l in range(7):
        agg = jnp.dot(m_acc, feat, preferred_element_type=jnp.float32)
        feat = _leaky(jnp.dot(agg, ws_ref[l], preferred_element_type=jnp.float32)
                      + bs_ref[l:l + 1, :])

    h = jnp.maximum(jnp.dot(feat, w1_ref[...], preferred_element_type=jnp.float32)
                    + b1_ref[...], 0.0)
    h = jnp.dot(h, w2_ref[...], preferred_element_type=jnp.float32) + b2_ref[...]

    gfeat = jnp.max(h, axis=0, keepdims=True)
    fc = jnp.maximum(jnp.dot(gfeat, fcw_ref[...], preferred_element_type=jnp.float32)
                     + fcb_ref[...], 0.0)
    t_out = jnp.dot(fc, tw_ref[...], preferred_element_type=jnp.float32) + tb_ref[...]

    aq = aq_ref[...]
    q = jnp.mean(aq, axis=0, keepdims=True)
    qn = q / (jnp.sqrt(jnp.sum(q * q)) + 1e-8)

    i = pl.program_id(0)
    out_ref[pl.ds(i, 1), :] = jnp.concatenate([qn, t_out, fc], axis=1)


@jax.jit
def kernel(x, params):
    b = x.shape[0]
    xt = jnp.swapaxes(x, 1, 2)
    convs = params["convs"]
    w0 = convs[0][0]
    b0 = convs[0][1].reshape(1, 32)
    ws = jnp.stack([w for w, _ in convs[1:]])
    bs = jnp.stack([bb for _, bb in convs[1:]])

    fixed = lambda *zeros: (lambda i: zeros)
    out = pl.pallas_call(
        _fwd_kernel,
        out_shape=jax.ShapeDtypeStruct((b, 71), jnp.float32),
        grid=(b,),
        in_specs=[
            pl.BlockSpec((1, _N, 3), lambda i: (i, 0, 0)),
            pl.BlockSpec((1, 3, _N), lambda i: (i, 0, 0)),
            pl.BlockSpec((1, 32), fixed(0, 0)),
            pl.BlockSpec((1, 32), fixed(0, 0)),
            pl.BlockSpec((7, 32, 32), fixed(0, 0, 0)),
            pl.BlockSpec((7, 32), fixed(0, 0)),
            pl.BlockSpec((32, 128), fixed(0, 0)),
            pl.BlockSpec((1, 128), fixed(0, 0)),
            pl.BlockSpec((128, 128), fixed(0, 0)),
            pl.BlockSpec((1, 128), fixed(0, 0)),
            pl.BlockSpec((_NA, 4), fixed(0, 0)),
            pl.BlockSpec((128, 64), fixed(0, 0)),
            pl.BlockSpec((1, 64), fixed(0, 0)),
            pl.BlockSpec((64, 3), fixed(0, 0)),
            pl.BlockSpec((1, 3), fixed(0, 0)),
        ],
        out_specs=pl.BlockSpec((b, 71), lambda i: (0, 0)),
    )(x, xt, w0, b0, ws, bs,
      params["out_W1"], params["out_b1"].reshape(1, 128),
      params["out_W2"], params["out_b2"].reshape(1, 128),
      params["anchor_quats"],
      params["fc_W"], params["fc_b"].reshape(1, 64),
      params["t_W"], params["t_b"].reshape(1, 3))
    return out
